# baseline (speedup 1.0000x reference)
import jax
import jax.numpy as jnp
from jax.experimental import pallas as pl
from jax.experimental.pallas import tpu as pltpu

EDGES = 131072
LANES = 128
ROWS = EDGES // LANES
BLK_ROWS = 128


def _edge_math(sp, tp, c, bas, scale):
    t0, t1, t2, x1, y1, z1, w1 = sp
    u0, u1, u2, x2, y2, z2, w2 = tp
    r, th, ph = c

    inv1 = 1.0 / jnp.maximum(jnp.sqrt(x1 * x1 + y1 * y1 + z1 * z1 + w1 * w1), 1e-12)
    x1, y1, z1, w1 = x1 * inv1, y1 * inv1, z1 * inv1, w1 * inv1
    inv2 = 1.0 / jnp.maximum(jnp.sqrt(x2 * x2 + y2 * y2 + z2 * z2 + w2 * w2), 1e-12)
    x2, y2, z2, w2 = x2 * inv2, y2 * inv2, z2 * inv2, w2 * inv2

    cph = jnp.cos(ph)
    px = r * cph * jnp.cos(th)
    py = r * cph * jnp.sin(th)
    pz = r * jnp.sin(ph)

    tx = 2.0 * (y1 * pz - z1 * py)
    ty = 2.0 * (z1 * px - x1 * pz)
    tz = 2.0 * (x1 * py - y1 * px)
    gx = px + w1 * tx + (y1 * tz - z1 * ty) + t0
    gy = py + w1 * ty + (z1 * tx - x1 * tz) + t1
    gz = pz + w1 * tz + (x1 * ty - y1 * tx) + t2

    dx, dy, dz = gx - u0, gy - u1, gz - u2
    tx = -2.0 * (y2 * dz - z2 * dy)
    ty = -2.0 * (z2 * dx - x2 * dz)
    tz = -2.0 * (x2 * dy - y2 * dx)
    lx = dx + w2 * tx - (y2 * tz - z2 * ty)
    ly = dy + w2 * ty - (z2 * tx - x2 * tz)
    lz = dz + w2 * tz - (x2 * ty - y2 * tx)

    r2 = jnp.sqrt(lx * lx + ly * ly + lz * lz)
    th2 = jnp.arctan2(ly, lx)

    b0, b1 = bas
    s0, s1 = scale
    return (r2 - b0) * s0, (th2 - b1) * s1


def _tc_body(g_ref, bas_ref, scale_ref, out_ref):
    sp = tuple(g_ref[i] for i in range(7))
    tp = tuple(g_ref[7 + i] for i in range(7))
    c = tuple(g_ref[14 + i] for i in range(3))
    bas = (bas_ref[0], bas_ref[1])
    s0 = scale_ref[0, 0]
    s1 = scale_ref[0, 1]
    ex, ey = _edge_math(sp, tp, c, bas, (s0, s1))
    out_ref[0] = ex
    out_ref[1] = ey


def _tc_math(g, bas, scale):
    n_blk = ROWS // BLK_ROWS
    return pl.pallas_call(
        _tc_body,
        grid=(n_blk,),
        in_specs=[
            pl.BlockSpec((17, BLK_ROWS, LANES), lambda i: (0, i, 0)),
            pl.BlockSpec((2, BLK_ROWS, LANES), lambda i: (0, i, 0)),
            pl.BlockSpec(memory_space=pltpu.SMEM),
        ],
        out_specs=pl.BlockSpec((2, BLK_ROWS, LANES), lambda i: (0, i, 0)),
        out_shape=jax.ShapeDtypeStruct((2, ROWS, LANES), jnp.float32),
    )(g, bas, scale)


def kernel(poses, patch_coords_r_theta, elevation_angle, coords_baseline,
           fls2physic_scale_factor, source_frame_idx, target_frame_idx, patch_idx):
    sp = poses[0, source_frame_idx, :]
    tp = poses[0, target_frame_idx, :]
    pc = patch_coords_r_theta[0, patch_idx, :]
    el = elevation_angle[0, patch_idx, :]
    g = jnp.concatenate([sp, tp, pc, el], axis=-1)
    g = g.T.reshape(17, ROWS, LANES)

    bas = coords_baseline[0].T.reshape(2, ROWS, LANES)
    scale = fls2physic_scale_factor.reshape(1, 2)

    out = _tc_math(g, bas, scale)
    return out.reshape(2, EDGES).T.reshape(1, 2 * EDGES)
